# baseline (speedup 1.0000x reference)
import functools

import jax
import jax.numpy as jnp
from jax import lax
from jax.experimental import pallas as pl
from jax.experimental.pallas import tpu as pltpu
from jax.experimental.pallas import tpu_sc as plsc

B, S, D = 4, 4096, 4096
NUM_EXPERTS = 64
K = 8
ROWS = B * S
BLK = 512

NC, NS, L = 2, 16, 16
NW = NC * NS
NCHUNK = 4
CHUNK = ROWS // NCHUNK
RPW = CHUNK // NW
GROUPS = RPW // L


def _router_block(x_ref, w_ref, probs_ref, probs_t_ref):
    s = jnp.dot(x_ref[...], w_ref[...], preferred_element_type=jnp.float32)
    m = jnp.max(s, axis=-1, keepdims=True)
    e = jnp.exp(s - m)
    p = e / jnp.sum(e, axis=-1, keepdims=True)
    probs_ref[...] = p
    probs_t_ref[...] = p.T


def _tc_router(xf, expert_embs):
    return pl.pallas_call(
        _router_block,
        grid=(CHUNK // BLK,),
        in_specs=[
            pl.BlockSpec((BLK, D), lambda i: (i, 0)),
            pl.BlockSpec((D, NUM_EXPERTS), lambda i: (0, 0)),
        ],
        out_specs=[
            pl.BlockSpec((BLK, NUM_EXPERTS), lambda i: (i, 0)),
            pl.BlockSpec((NUM_EXPERTS, BLK), lambda i: (0, i)),
        ],
        out_shape=[
            jax.ShapeDtypeStruct((CHUNK, NUM_EXPERTS), jnp.float32),
            jax.ShapeDtypeStruct((NUM_EXPERTS, CHUNK), jnp.float32),
        ],
    )(xf, expert_embs)


def _sc_topk_body(probs_t_hbm, out_hbm, in_v, out_v):
    wid = lax.axis_index("s") * NC + lax.axis_index("c")
    base = wid * RPW
    pltpu.sync_copy(probs_t_hbm.at[:, pl.ds(base, RPW)], in_v)

    neg = jnp.full((L,), -jnp.inf, jnp.float32)

    def group(g, carry):
        lr = g * L
        tops = [neg] * K
        for e in range(NUM_EXPERTS):
            r = in_v[e, pl.ds(lr, L)]
            for j in range(K):
                hi = jnp.maximum(tops[j], r)
                r = jnp.minimum(tops[j], r)
                tops[j] = hi
        thresh = tops[K - 1]
        for e in range(NUM_EXPERTS):
            v = in_v[e, pl.ds(lr, L)]
            out_v[e, pl.ds(lr, L)] = jnp.where(v >= thresh, v, 0.0)
        return carry

    lax.fori_loop(0, GROUPS, group, 0)
    pltpu.sync_copy(out_v, out_hbm.at[:, pl.ds(base, RPW)])


@functools.partial(
    pl.kernel,
    mesh=plsc.VectorSubcoreMesh(core_axis_name="c", subcore_axis_name="s"),
    compiler_params=pltpu.CompilerParams(needs_layout_passes=False),
    out_type=jax.ShapeDtypeStruct((NUM_EXPERTS, CHUNK), jnp.float32),
    scratch_types=[
        pltpu.VMEM((NUM_EXPERTS, RPW), jnp.float32),
        pltpu.VMEM((NUM_EXPERTS, RPW), jnp.float32),
    ],
)
def _sc_topk(probs_t_hbm, out_hbm, in_v, out_v):
    _sc_topk_body(probs_t_hbm, out_hbm, in_v, out_v)


def kernel(x, expert_embs):
    xf = x.reshape(ROWS, D)
    probs_chunks, masks_chunks = [], []
    for c in range(NCHUNK):
        p_c, pt_c = _tc_router(
            lax.slice(xf, (c * CHUNK, 0), ((c + 1) * CHUNK, D)), expert_embs)
        probs_chunks.append(p_c)
        masks_chunks.append(_sc_topk(pt_c))
    probs = jnp.concatenate(probs_chunks, axis=0)
    masks_t = jnp.concatenate(masks_chunks, axis=1)
    experts_masks = masks_t.reshape(NUM_EXPERTS, B, S, 1)
    aux_loss = jnp.zeros((), jnp.float32)
    return (experts_masks, aux_loss, probs)


# speedup vs baseline: 2.1010x; 2.1010x over previous
import functools

import jax
import jax.numpy as jnp
from jax import lax
from jax.experimental import pallas as pl
from jax.experimental.pallas import tpu as pltpu
from jax.experimental.pallas import tpu_sc as plsc

B, S, D = 4, 4096, 4096
NUM_EXPERTS = 64
K = 8
ROWS = B * S
BLK = 512

NC, NS, L = 2, 16, 16
NW = NC * NS
NCHUNK = 4
CHUNK = ROWS // NCHUNK
RPW = CHUNK // NW
GROUPS = RPW // L


def _router_block(x_ref, w_ref, probs_ref, probs_t_ref):
    s = jnp.dot(x_ref[...], w_ref[...], preferred_element_type=jnp.float32)
    m = jnp.max(s, axis=-1, keepdims=True)
    e = jnp.exp(s - m)
    p = e / jnp.sum(e, axis=-1, keepdims=True)
    probs_ref[...] = p
    probs_t_ref[...] = p.T


def _tc_router(xf, expert_embs, c):
    off = c * (CHUNK // BLK)
    return pl.pallas_call(
        _router_block,
        grid=(CHUNK // BLK,),
        in_specs=[
            pl.BlockSpec((BLK, D), lambda i: (off + i, 0)),
            pl.BlockSpec((D, NUM_EXPERTS), lambda i: (0, 0)),
        ],
        out_specs=[
            pl.BlockSpec((BLK, NUM_EXPERTS), lambda i: (i, 0)),
            pl.BlockSpec((NUM_EXPERTS, BLK), lambda i: (0, i)),
        ],
        out_shape=[
            jax.ShapeDtypeStruct((CHUNK, NUM_EXPERTS), jnp.float32),
            jax.ShapeDtypeStruct((NUM_EXPERTS, CHUNK), jnp.float32),
        ],
    )(xf, expert_embs)


def _sc_topk_body(probs_t_hbm, out_hbm, in_v, out_v):
    wid = lax.axis_index("s") * NC + lax.axis_index("c")
    base = wid * RPW
    pltpu.sync_copy(probs_t_hbm.at[:, pl.ds(base, RPW)], in_v)

    neg = jnp.full((L,), -jnp.inf, jnp.float32)

    def group(g, carry):
        lr = g * L
        tops = [neg] * K
        for e in range(NUM_EXPERTS):
            r = in_v[e, pl.ds(lr, L)]
            for j in range(K):
                hi = jnp.maximum(tops[j], r)
                r = jnp.minimum(tops[j], r)
                tops[j] = hi
        thresh = tops[K - 1]
        for e in range(NUM_EXPERTS):
            v = in_v[e, pl.ds(lr, L)]
            out_v[e, pl.ds(lr, L)] = jnp.where(v >= thresh, v, 0.0)
        return carry

    lax.fori_loop(0, GROUPS, group, 0)
    pltpu.sync_copy(out_v, out_hbm.at[:, pl.ds(base, RPW)])


@functools.partial(
    pl.kernel,
    mesh=plsc.VectorSubcoreMesh(core_axis_name="c", subcore_axis_name="s"),
    compiler_params=pltpu.CompilerParams(needs_layout_passes=False),
    out_type=jax.ShapeDtypeStruct((NUM_EXPERTS, CHUNK), jnp.float32),
    scratch_types=[
        pltpu.VMEM((NUM_EXPERTS, RPW), jnp.float32),
        pltpu.VMEM((NUM_EXPERTS, RPW), jnp.float32),
    ],
)
def _sc_topk(probs_t_hbm, out_hbm, in_v, out_v):
    _sc_topk_body(probs_t_hbm, out_hbm, in_v, out_v)


def kernel(x, expert_embs):
    xf = x.reshape(ROWS, D)
    probs_chunks, masks_chunks = [], []
    for c in range(NCHUNK):
        p_c, pt_c = _tc_router(xf, expert_embs, c)
        probs_chunks.append(p_c)
        masks_chunks.append(_sc_topk(pt_c))
    probs = jnp.concatenate(probs_chunks, axis=0)
    masks_t = jnp.concatenate(masks_chunks, axis=1)
    experts_masks = masks_t.reshape(NUM_EXPERTS, B, S, 1)
    aux_loss = jnp.zeros((), jnp.float32)
    return (experts_masks, aux_loss, probs)


# speedup vs baseline: 2.2766x; 1.0836x over previous
import functools

import jax
import jax.numpy as jnp
from jax import lax
from jax.experimental import pallas as pl
from jax.experimental.pallas import tpu as pltpu
from jax.experimental.pallas import tpu_sc as plsc

B, S, D = 4, 4096, 4096
NUM_EXPERTS = 64
K = 8
ROWS = B * S
BLK = 1024

NC, NS, L = 2, 16, 16
NW = NC * NS
NCHUNK = 1
CHUNK = ROWS // NCHUNK
RPW = CHUNK // NW
GROUPS = RPW // L


def _router_block(x_ref, w_ref, probs_ref, probs_t_ref):
    s = jnp.dot(x_ref[...], w_ref[...], preferred_element_type=jnp.float32)
    m = jnp.max(s, axis=-1, keepdims=True)
    e = jnp.exp(s - m)
    p = e / jnp.sum(e, axis=-1, keepdims=True)
    probs_ref[...] = p
    probs_t_ref[...] = p.T


def _tc_router(xf, expert_embs, c):
    off = c * (CHUNK // BLK)
    return pl.pallas_call(
        _router_block,
        grid=(CHUNK // BLK,),
        in_specs=[
            pl.BlockSpec((BLK, D), lambda i: (off + i, 0)),
            pl.BlockSpec((D, NUM_EXPERTS), lambda i: (0, 0)),
        ],
        out_specs=[
            pl.BlockSpec((BLK, NUM_EXPERTS), lambda i: (i, 0)),
            pl.BlockSpec((NUM_EXPERTS, BLK), lambda i: (0, i)),
        ],
        out_shape=[
            jax.ShapeDtypeStruct((CHUNK, NUM_EXPERTS), jnp.float32),
            jax.ShapeDtypeStruct((NUM_EXPERTS, CHUNK), jnp.float32),
        ],
    )(xf, expert_embs)


def _sc_topk_body(probs_t_hbm, out_hbm, in_v, out_v):
    wid = lax.axis_index("s") * NC + lax.axis_index("c")
    base = wid * RPW
    pltpu.sync_copy(probs_t_hbm.at[:, pl.ds(base, RPW)], in_v)

    neg = jnp.full((L,), -jnp.inf, jnp.float32)

    def group(g, carry):
        lr = g * L
        tops = [neg] * K
        for e in range(NUM_EXPERTS):
            r = in_v[e, pl.ds(lr, L)]
            for j in range(K):
                hi = jnp.maximum(tops[j], r)
                r = jnp.minimum(tops[j], r)
                tops[j] = hi
        thresh = tops[K - 1]
        for e in range(NUM_EXPERTS):
            v = in_v[e, pl.ds(lr, L)]
            out_v[e, pl.ds(lr, L)] = jnp.where(v >= thresh, v, 0.0)
        return carry

    lax.fori_loop(0, GROUPS, group, 0)
    pltpu.sync_copy(out_v, out_hbm.at[:, pl.ds(base, RPW)])


@functools.partial(
    pl.kernel,
    mesh=plsc.VectorSubcoreMesh(core_axis_name="c", subcore_axis_name="s"),
    compiler_params=pltpu.CompilerParams(needs_layout_passes=False),
    out_type=jax.ShapeDtypeStruct((NUM_EXPERTS, CHUNK), jnp.float32),
    scratch_types=[
        pltpu.VMEM((NUM_EXPERTS, RPW), jnp.float32),
        pltpu.VMEM((NUM_EXPERTS, RPW), jnp.float32),
    ],
)
def _sc_topk(probs_t_hbm, out_hbm, in_v, out_v):
    _sc_topk_body(probs_t_hbm, out_hbm, in_v, out_v)


def kernel(x, expert_embs):
    xf = x.reshape(ROWS, D)
    probs_chunks, masks_chunks = [], []
    for c in range(NCHUNK):
        p_c, pt_c = _tc_router(xf, expert_embs, c)
        probs_chunks.append(p_c)
        masks_chunks.append(_sc_topk(pt_c))
    probs = jnp.concatenate(probs_chunks, axis=0)
    masks_t = jnp.concatenate(masks_chunks, axis=1)
    experts_masks = masks_t.reshape(NUM_EXPERTS, B, S, 1)
    aux_loss = jnp.zeros((), jnp.float32)
    return (experts_masks, aux_loss, probs)
